# baseline (speedup 1.0000x reference)
import jax
import jax.numpy as jnp
from jax import lax
from jax.experimental import pallas as pl
from jax.experimental.pallas import tpu as pltpu
from jax.experimental.pallas import tpu_sc as plsc

_BATCH = 16384
_IN_COLS = 40
_OUT_COLS = 69
_LANES = 16
_NUM_WORKERS = 32
_ROWS_PER = _BATCH // _NUM_WORKERS


def _sc_body(in_hbm, w0_hbm, w1_hbm, w2_hbm, out_hbm,
             in_v, out_v, w0_v, w1_v, w2_v):
    wid = lax.axis_index("s") * 2 + lax.axis_index("c")

    pltpu.sync_copy(in_hbm.at[pl.ds(wid * _ROWS_PER * _IN_COLS,
                                    _ROWS_PER * _IN_COLS)], in_v)
    pltpu.sync_copy(w0_hbm, w0_v)
    pltpu.sync_copy(w1_hbm, w1_v)
    pltpu.sync_copy(w2_hbm, w2_v)

    lanes = lax.iota(jnp.int32, _LANES)

    def group(g, carry):
        rows = g * _LANES + lanes
        rin = rows * _IN_COLS
        rout = rows * _OUT_COLS
        b0 = jnp.minimum(
            plsc.load_gather(in_v, [rin]).astype(jnp.int32), 64) * 16
        b1 = jnp.minimum(
            plsc.load_gather(in_v, [rin + 1]).astype(jnp.int32), 32) * 8
        b2 = jnp.minimum(
            plsc.load_gather(in_v, [rin + 2]).astype(jnp.int32), 16) * 8
        for c in range(37):
            v = plsc.load_gather(in_v, [rin + (c + 3)])
            plsc.store_scatter(out_v, [rout + c], v)
        for c in range(16):
            v = plsc.load_gather(w0_v, [b0 + c])
            plsc.store_scatter(out_v, [rout + (37 + c)], v)
        for c in range(8):
            v = plsc.load_gather(w1_v, [b1 + c])
            plsc.store_scatter(out_v, [rout + (53 + c)], v)
        for c in range(8):
            v = plsc.load_gather(w2_v, [b2 + c])
            plsc.store_scatter(out_v, [rout + (61 + c)], v)
        return carry

    lax.fori_loop(0, _ROWS_PER // _LANES, group, 0)
    pltpu.sync_copy(out_v, out_hbm.at[pl.ds(wid * _ROWS_PER * _OUT_COLS,
                                            _ROWS_PER * _OUT_COLS)])


def kernel(inputs, W0, W1, W2):
    mesh = plsc.VectorSubcoreMesh(core_axis_name="c", subcore_axis_name="s")
    fn = pl.kernel(
        _sc_body,
        out_type=jax.ShapeDtypeStruct((_BATCH * _OUT_COLS,), jnp.float32),
        mesh=mesh,
        scratch_types=[
            pltpu.VMEM((_ROWS_PER * _IN_COLS,), jnp.float32),
            pltpu.VMEM((_ROWS_PER * _OUT_COLS,), jnp.float32),
            pltpu.VMEM((65 * 16,), jnp.float32),
            pltpu.VMEM((33 * 8,), jnp.float32),
            pltpu.VMEM((17 * 8,), jnp.float32),
        ],
        compiler_params=pltpu.CompilerParams(needs_layout_passes=False),
    )
    out = fn(inputs.reshape(-1), W0.reshape(-1), W1.reshape(-1),
             W2.reshape(-1))
    return out.reshape(_BATCH, _OUT_COLS)


# speedup vs baseline: 1.0587x; 1.0587x over previous
import jax
import jax.numpy as jnp
from jax import lax
from jax.experimental import pallas as pl
from jax.experimental.pallas import tpu as pltpu
from jax.experimental.pallas import tpu_sc as plsc

_BATCH = 16384
_IN_COLS = 40
_OUT_COLS = 69
_LANES = 16
_NUM_WORKERS = 32
_ROWS_PER = _BATCH // _NUM_WORKERS
_CHUNK = 256
_GROUPS = _CHUNK // _LANES


def _splat(c):
    return jnp.full((_LANES,), c, jnp.int32)


def _sc_body(in_hbm, w0_hbm, w1_hbm, w2_hbm, out_hbm,
             in_v, out_v, w0_v, w1_v, w2_v):
    wid = lax.axis_index("s") * 2 + lax.axis_index("c")
    pltpu.sync_copy(w0_hbm, w0_v)
    pltpu.sync_copy(w1_hbm, w1_v)
    pltpu.sync_copy(w2_hbm, w2_v)

    lanes = lax.iota(jnp.int32, _LANES)

    for chunk in range(_ROWS_PER // _CHUNK):
        base = wid * _ROWS_PER + chunk * _CHUNK
        pltpu.sync_copy(in_hbm.at[pl.ds(base, _CHUNK)], in_v)

        @plsc.parallel_loop(0, _GROUPS, unroll=2)
        def group(g):
            rvec = g * _LANES + lanes
            b0 = jnp.minimum(
                plsc.load_gather(in_v, [rvec, _splat(0)]).astype(jnp.int32),
                64) * 16
            b1 = jnp.minimum(
                plsc.load_gather(in_v, [rvec, _splat(1)]).astype(jnp.int32),
                32) * 8
            b2 = jnp.minimum(
                plsc.load_gather(in_v, [rvec, _splat(2)]).astype(jnp.int32),
                16) * 8
            for c in range(37):
                v = plsc.load_gather(in_v, [rvec, _splat(c + 3)])
                plsc.store_scatter(out_v, [rvec, _splat(c)], v)
            for c in range(16):
                v = plsc.load_gather(w0_v, [b0 + c])
                plsc.store_scatter(out_v, [rvec, _splat(37 + c)], v)
            for c in range(8):
                v = plsc.load_gather(w1_v, [b1 + c])
                plsc.store_scatter(out_v, [rvec, _splat(53 + c)], v)
            for c in range(8):
                v = plsc.load_gather(w2_v, [b2 + c])
                plsc.store_scatter(out_v, [rvec, _splat(61 + c)], v)

        pltpu.sync_copy(out_v, out_hbm.at[pl.ds(base, _CHUNK)])


def kernel(inputs, W0, W1, W2):
    mesh = plsc.VectorSubcoreMesh(core_axis_name="c", subcore_axis_name="s")
    fn = pl.kernel(
        _sc_body,
        out_type=jax.ShapeDtypeStruct((_BATCH, _OUT_COLS), jnp.float32),
        mesh=mesh,
        scratch_types=[
            pltpu.VMEM((_CHUNK, _IN_COLS), jnp.float32),
            pltpu.VMEM((_CHUNK, _OUT_COLS), jnp.float32),
            pltpu.VMEM((65 * 16,), jnp.float32),
            pltpu.VMEM((33 * 8,), jnp.float32),
            pltpu.VMEM((17 * 8,), jnp.float32),
        ],
        compiler_params=pltpu.CompilerParams(needs_layout_passes=False),
    )
    return fn(inputs, W0.reshape(-1), W1.reshape(-1), W2.reshape(-1))


# speedup vs baseline: 1.1246x; 1.0622x over previous
import jax
import jax.numpy as jnp
from jax import lax
from jax.experimental import pallas as pl
from jax.experimental.pallas import tpu as pltpu
from jax.experimental.pallas import tpu_sc as plsc

_BATCH = 16384
_IN_COLS = 40
_OUT_COLS = 69
_LANES = 16
_NUM_WORKERS = 32
_ROWS_PER = _BATCH // _NUM_WORKERS
_CHUNK = 256
_GROUPS = _CHUNK // _LANES


def _splat(c):
    return jnp.full((_LANES,), c, jnp.int32)


def _sc_body(in_hbm, w0_hbm, w1_hbm, w2_hbm, out_hbm,
             in_v, out_v, w0_v, w1_v, w2_v):
    wid = lax.axis_index("s") * 2 + lax.axis_index("c")
    pltpu.sync_copy(w0_hbm, w0_v)
    pltpu.sync_copy(w1_hbm, w1_v)
    pltpu.sync_copy(w2_hbm, w2_v)

    lanes = lax.iota(jnp.int32, _LANES)

    for chunk in range(_ROWS_PER // _CHUNK):
        base = wid * _ROWS_PER + chunk * _CHUNK
        pltpu.sync_copy(in_hbm.at[pl.ds(base, _CHUNK)], in_v)

        @plsc.parallel_loop(0, _GROUPS, unroll=4)
        def group(g):
            rvec = g * _LANES + lanes
            b0 = jnp.minimum(
                plsc.load_gather(in_v, [rvec, _splat(0)]).astype(jnp.int32),
                64) * 16
            b1 = jnp.minimum(
                plsc.load_gather(in_v, [rvec, _splat(1)]).astype(jnp.int32),
                32) * 8
            b2 = jnp.minimum(
                plsc.load_gather(in_v, [rvec, _splat(2)]).astype(jnp.int32),
                16) * 8
            for c in range(37):
                v = plsc.load_gather(in_v, [rvec, _splat(c + 3)])
                plsc.store_scatter(out_v, [rvec, _splat(c)], v)
            for c in range(16):
                v = plsc.load_gather(w0_v, [b0 + c])
                plsc.store_scatter(out_v, [rvec, _splat(37 + c)], v)
            for c in range(8):
                v = plsc.load_gather(w1_v, [b1 + c])
                plsc.store_scatter(out_v, [rvec, _splat(53 + c)], v)
            for c in range(8):
                v = plsc.load_gather(w2_v, [b2 + c])
                plsc.store_scatter(out_v, [rvec, _splat(61 + c)], v)

        pltpu.sync_copy(out_v, out_hbm.at[pl.ds(base, _CHUNK)])


def kernel(inputs, W0, W1, W2):
    mesh = plsc.VectorSubcoreMesh(core_axis_name="c", subcore_axis_name="s")
    fn = pl.kernel(
        _sc_body,
        out_type=jax.ShapeDtypeStruct((_BATCH, _OUT_COLS), jnp.float32),
        mesh=mesh,
        scratch_types=[
            pltpu.VMEM((_CHUNK, _IN_COLS), jnp.float32),
            pltpu.VMEM((_CHUNK, _OUT_COLS), jnp.float32),
            pltpu.VMEM((65 * 16,), jnp.float32),
            pltpu.VMEM((33 * 8,), jnp.float32),
            pltpu.VMEM((17 * 8,), jnp.float32),
        ],
        compiler_params=pltpu.CompilerParams(needs_layout_passes=False),
    )
    return fn(inputs, W0.reshape(-1), W1.reshape(-1), W2.reshape(-1))


# speedup vs baseline: 1.7001x; 1.5117x over previous
import jax
import jax.numpy as jnp
from jax import lax
from jax.experimental import pallas as pl
from jax.experimental.pallas import tpu as pltpu
from jax.experimental.pallas import tpu_sc as plsc

_BATCH = 16384
_IN_COLS = 40
_OUT_COLS = 69
_LANES = 16
_NUM_WORKERS = 32
_ROWS_PER = _BATCH // _NUM_WORKERS
_CHUNK = 256
_W12_LEN = 33 * 8 + 17 * 8


def _sc_body(in_hbm, w0_hbm, w12_hbm, out_hbm, in_v, out_v, w0_v, w12_v):
    wid = lax.axis_index("s") * 2 + lax.axis_index("c")
    pltpu.sync_copy(w0_hbm, w0_v)
    pltpu.sync_copy(w12_hbm, w12_v)

    lanes = lax.iota(jnp.int32, _LANES)
    lo = lanes < 8
    off12 = jnp.where(lo, lanes, (33 * 8 - 8) + lanes)

    for chunk in range(_ROWS_PER // _CHUNK):
        base = wid * _ROWS_PER + chunk * _CHUNK
        pltpu.sync_copy(in_hbm.at[pl.ds(base, _CHUNK)], in_v)

        @plsc.parallel_loop(0, _CHUNK, unroll=4)
        def row(r):
            ids = in_v[r, pl.ds(0, 16)].astype(jnp.int32)
            b0 = jnp.minimum(ids[0], 64) * 16
            b1 = jnp.minimum(ids[1], 32) * 8
            b2 = jnp.minimum(ids[2], 16) * 8
            out_v[r, pl.ds(0, 16)] = in_v[r, pl.ds(3, 16)]
            out_v[r, pl.ds(16, 16)] = in_v[r, pl.ds(19, 16)]
            out_v[r, pl.ds(21, 16)] = in_v[r, pl.ds(24, 16)]
            out_v[r, pl.ds(37, 16)] = w0_v[pl.ds(b0, 16)]
            idx12 = jnp.where(lo, b1, b2) + off12
            out_v[r, pl.ds(53, 16)] = plsc.load_gather(w12_v, [idx12])

        pltpu.sync_copy(out_v, out_hbm.at[pl.ds(base, _CHUNK)])


def kernel(inputs, W0, W1, W2):
    mesh = plsc.VectorSubcoreMesh(core_axis_name="c", subcore_axis_name="s")
    fn = pl.kernel(
        _sc_body,
        out_type=jax.ShapeDtypeStruct((_BATCH, _OUT_COLS), jnp.float32),
        mesh=mesh,
        scratch_types=[
            pltpu.VMEM((_CHUNK, _IN_COLS), jnp.float32),
            pltpu.VMEM((_CHUNK, _OUT_COLS), jnp.float32),
            pltpu.VMEM((65 * 16,), jnp.float32),
            pltpu.VMEM((_W12_LEN,), jnp.float32),
        ],
        compiler_params=pltpu.CompilerParams(needs_layout_passes=False),
    )
    w12 = jnp.concatenate([W1.reshape(-1), W2.reshape(-1)])
    return fn(inputs, W0.reshape(-1), w12)
